# baseline (speedup 1.0000x reference)
import functools

import jax
import jax.numpy as jnp
from jax import lax
from jax.experimental import pallas as pl
from jax.experimental.pallas import tpu as pltpu
from jax.experimental.pallas import tpu_sc as plsc

B = 16384
VISION_DIM = 768
PROMPT_DIM = 768
NUM_PROMPTS = 1024
TOP_K = 8

BM = 2048

NW = 32
RPW = B // NW
RCHUNK = 4
NCHUNK = RPW // RCHUNK
LANES = 16


def _front_body(vf_ref, wt_ref, b_ref, idx_ref, w_ref):
    s0 = (
        jnp.dot(vf_ref[...], wt_ref[...], preferred_element_type=jnp.float32)
        + b_ref[...]
    )
    col = lax.broadcasted_iota(jnp.int32, (BM, NUM_PROMPTS), 1)
    vals = []
    idxs = []
    s = s0
    for k in range(TOP_K):
        m = jnp.max(s, axis=1, keepdims=True)
        hit = s == m
        vals.append(m)
        idxs.append(jnp.min(jnp.where(hit, col, NUM_PROMPTS), axis=1, keepdims=True))
        if k < TOP_K - 1:
            s = jnp.where(hit, -jnp.inf, s)
    v = jnp.concatenate(vals, axis=1)
    e = jnp.exp(v - vals[0])
    w = e / jnp.sum(e, axis=1, keepdims=True)
    idx_ref[...] = jnp.concatenate(idxs, axis=1)
    w_ref[...] = w


def _tc_front(vision_features, wt, b2):
    nrows = vision_features.shape[0]
    grid = (nrows // BM,)
    return pl.pallas_call(
        _front_body,
        grid=grid,
        in_specs=[
            pl.BlockSpec((BM, VISION_DIM), lambda i: (i, 0)),
            pl.BlockSpec((VISION_DIM, NUM_PROMPTS), lambda i: (0, 0)),
            pl.BlockSpec((1, NUM_PROMPTS), lambda i: (0, 0)),
        ],
        out_specs=[
            pl.BlockSpec((BM, TOP_K), lambda i: (i, 0)),
            pl.BlockSpec((BM, TOP_K), lambda i: (i, 0)),
        ],
        out_shape=[
            jax.ShapeDtypeStruct((nrows, TOP_K), jnp.int32),
            jax.ShapeDtypeStruct((nrows, TOP_K), jnp.float32),
        ],
        compiler_params=pltpu.CompilerParams(
            dimension_semantics=("parallel",),
        ),
    )(vision_features, wt, b2)


_SC_MESH = plsc.VectorSubcoreMesh(core_axis_name="c", subcore_axis_name="s")


@functools.lru_cache(maxsize=None)
def _make_sc_combine(nrows):
  rpw = nrows // NW
  nchunk = rpw // RCHUNK

  @functools.partial(
      pl.kernel,
      mesh=_SC_MESH,
      out_type=jax.ShapeDtypeStruct((nrows, PROMPT_DIM), jnp.float32),
      scratch_types=[
          pltpu.VMEM((rpw * TOP_K,), jnp.int32),
          pltpu.VMEM((rpw * TOP_K,), jnp.float32),
          pltpu.VMEM((RCHUNK * TOP_K, PROMPT_DIM), jnp.float32),
          pltpu.VMEM((RCHUNK * TOP_K, PROMPT_DIM), jnp.float32),
          pltpu.VMEM((RCHUNK, PROMPT_DIM), jnp.float32),
          pltpu.VMEM((RCHUNK, PROMPT_DIM), jnp.float32),
          pltpu.SemaphoreType.DMA,
          pltpu.SemaphoreType.DMA,
          pltpu.SemaphoreType.DMA,
          pltpu.SemaphoreType.DMA,
      ],
  )
  def _sc_combine(pool_hbm, idx_hbm, w_hbm, out_hbm,
                  idx_v, w_v, rows0, rows1, outb0, outb1,
                  sg0, sg1, so0, so1):
    RPW = rpw
    NCHUNK = nchunk
    wid = lax.axis_index("s") * 2 + lax.axis_index("c")
    base = wid * RPW
    rows_bufs = (rows0, rows1)
    out_bufs = (outb0, outb1)
    gsems = (sg0, sg1)
    osems = (so0, so1)

    pltpu.sync_copy(idx_hbm.at[pl.ds(base * TOP_K, RPW * TOP_K)], idx_v)
    pltpu.sync_copy(w_hbm.at[pl.ds(base * TOP_K, RPW * TOP_K)], w_v)

    def gather_start(g, b):
        idx_slice = idx_v.at[pl.ds(g * RCHUNK * TOP_K, RCHUNK * TOP_K)]
        pltpu.async_copy(pool_hbm.at[idx_slice], rows_bufs[b], gsems[b])

    gather_start(0, 0)
    gather_start(1, 1)

    def loop_body(i2, carry):
        for b in range(2):
            g = i2 * 2 + b
            pltpu.make_async_copy(
                pool_hbm.at[idx_v.at[pl.ds(0, RCHUNK * TOP_K)]],
                rows_bufs[b], gsems[b]).wait()

            @pl.when(i2 > 0)
            def _():
                pltpu.make_async_copy(
                    out_bufs[b], out_hbm.at[pl.ds(base, RCHUNK)], osems[b]
                ).wait()

            wv0 = w_v[pl.ds(g * RCHUNK * TOP_K, LANES)]
            wv1 = w_v[pl.ds(g * RCHUNK * TOP_K + LANES, LANES)]
            wvs = (wv0, wv1)
            for r in range(RCHUNK):
                wspl = [
                    wvs[r // 2].at[
                        jnp.full((LANES,), (r % 2) * TOP_K + k, jnp.int32)
                    ].get(mode="promise_in_bounds")
                    for k in range(TOP_K)
                ]
                for c in range(PROMPT_DIM // LANES):
                    acc = jnp.zeros((LANES,), jnp.float32)
                    for k in range(TOP_K):
                        acc = acc + wspl[k] * rows_bufs[b][
                            r * TOP_K + k, pl.ds(c * LANES, LANES)
                        ]
                    out_bufs[b][r, pl.ds(c * LANES, LANES)] = acc

            pltpu.async_copy(
                out_bufs[b],
                out_hbm.at[pl.ds(base + g * RCHUNK, RCHUNK)],
                osems[b])

            @pl.when(g + 2 < NCHUNK)
            def _():
                gather_start(g + 2, b)
        return carry

    lax.fori_loop(0, NCHUNK // 2, loop_body, 0)
    for b in range(2):
        pltpu.make_async_copy(
            out_bufs[b], out_hbm.at[pl.ds(base, RCHUNK)], osems[b]).wait()

  return _sc_combine


@jax.jit
def kernel(vision_features, W, b, prompt_pool):
    wt = W.T
    b2 = b.reshape(1, NUM_PROMPTS)
    outs = []
    half = B // 2
    for h in range(2):
        vfh = lax.slice_in_dim(vision_features, h * half, (h + 1) * half, axis=0)
        idx, w = _tc_front(vfh, wt, b2)
        outs.append(
            _make_sc_combine(half)(prompt_pool, idx.reshape(half * TOP_K),
                                   w.reshape(half * TOP_K))
        )
    return jnp.concatenate(outs, axis=0)


# speedup vs baseline: 5.2856x; 5.2856x over previous
import jax
import jax.numpy as jnp
from jax.experimental import pallas as pl
from jax.experimental.pallas import tpu as pltpu

B = 16384
VISION_DIM = 768
PROMPT_DIM = 768
NUM_PROMPTS = 1024
TOP_K = 8

BM = 2048


def _body(vf_ref, wt_ref, b_ref, pool_ref, out_ref):
    s0 = (
        jnp.dot(vf_ref[...], wt_ref[...], preferred_element_type=jnp.float32)
        + b_ref[...]
    )
    m1 = jnp.max(s0, axis=1, keepdims=True)
    s = s0
    m = m1
    for _ in range(TOP_K - 1):
        s = jnp.where(s == m, -jnp.inf, s)
        m = jnp.max(s, axis=1, keepdims=True)
    e = jnp.exp(s0 - jnp.where(s0 >= m, m1, jnp.inf))
    z = jnp.sum(e, axis=1, keepdims=True)
    acc = jnp.dot(e, pool_ref[...], preferred_element_type=jnp.float32)
    out_ref[...] = acc * (1.0 / z)


@jax.jit
def kernel(vision_features, W, b, prompt_pool):
    wt = W.T
    b2 = b.reshape(1, NUM_PROMPTS)
    grid = (B // BM,)
    return pl.pallas_call(
        _body,
        grid=grid,
        in_specs=[
            pl.BlockSpec((BM, VISION_DIM), lambda i: (i, 0)),
            pl.BlockSpec((VISION_DIM, NUM_PROMPTS), lambda i: (0, 0)),
            pl.BlockSpec((1, NUM_PROMPTS), lambda i: (0, 0)),
            pl.BlockSpec((NUM_PROMPTS, PROMPT_DIM), lambda i: (0, 0)),
        ],
        out_specs=pl.BlockSpec((BM, PROMPT_DIM), lambda i: (i, 0)),
        out_shape=jax.ShapeDtypeStruct((B, PROMPT_DIM), jnp.float32),
        compiler_params=pltpu.CompilerParams(
            dimension_semantics=("parallel",),
        ),
    )(vision_features, wt, b2, prompt_pool)
